# baseline (speedup 1.0000x reference)
import functools

import jax
import jax.numpy as jnp
from jax import lax
from jax.experimental import pallas as pl
from jax.experimental.pallas import tpu as pltpu
from jax.experimental.pallas import tpu_sc as plsc

_B, _L, _D = 4096, 200, 64
_NC, _NS = 2, 16
_NW = _NC * _NS
_NBLK = _L * (_B // 128)
_PER_W = _NBLK // _NW
_BH = _B // 128

_mesh = plsc.VectorSubcoreMesh(core_axis_name="c", subcore_axis_name="s")


@functools.partial(
    pl.kernel,
    mesh=_mesh,
    out_type=jax.ShapeDtypeStruct((_L, 8, _BH, 8, 128), jnp.float32),
    scratch_types=[
        pltpu.VMEM((_PER_W, 128), jnp.int32),
        pltpu.VMEM((_PER_W, 128), jnp.int32),
        pltpu.VMEM((2, 128, 128), jnp.float32),
        pltpu.VMEM((2, 8, 8, 128), jnp.float32),
        pltpu.SemaphoreType.DMA,
        pltpu.SemaphoreType.DMA,
        pltpu.SemaphoreType.DMA,
    ],
    compiler_params=pltpu.CompilerParams(use_tc_tiling_on_sc=True,
                                         needs_layout_passes=False),
)
def _emb(idx_hbm, tab_hbm, out_hbm, pidx_v, pov_v, rows_v, trows_v, gsem,
         ssem0, ssem1):
    wid = lax.axis_index("s") * _NC + lax.axis_index("c")
    base = wid * _PER_W
    pltpu.sync_copy(idx_hbm.at[wid], pidx_v)

    @pl.loop(0, _PER_W)
    def _(t):
        for j in range(8):
            v = pidx_v[t, pl.ds(j * 16, 16)]
            pov_v[t, pl.ds(j * 16, 16)] = (v & 1) * 64
            pidx_v[t, pl.ds(j * 16, 16)] = v >> 1

    ssems = (ssem0, ssem1)
    lanes = lax.iota(jnp.int32, 16)
    ris = [b0 * 16 + lanes for b0 in range(8)]
    rots = [(lanes + k) % 16 for k in range(16)]

    def fire_g(t, h):
        pltpu.async_copy(tab_hbm.at[pidx_v.at[t]], rows_v.at[h], gsem)

    def drain_g(h):
        pltpu.make_async_copy(tab_hbm.at[pidx_v.at[0]], rows_v.at[h],
                              gsem).wait()

    def transpose(t, h):
        rv = rows_v.at[h]
        tv = trows_v.at[h]
        povs = [pov_v[t, pl.ds(b0 * 16, 16)] for b0 in range(8)]

        @pl.loop(0, 4, unroll=2)
        def _(q):
            d0 = q * 16
            for k in range(16):
                cik = rots[k] + d0
                ih = cik >> 3
                il = cik & 7
                for b0 in range(8):
                    v = plsc.load_gather(rv, [ris[b0], cik + povs[b0]])
                    plsc.store_scatter(tv, [ih, il, ris[b0]], v)

    def fire_s(t, h):
        blk = base + t
        l = blk // _BH
        bh = blk % _BH
        pltpu.async_copy(trows_v.at[h], out_hbm.at[l, :, bh], ssems[h])

    def drain_s(h):
        pltpu.make_async_copy(trows_v.at[h], out_hbm.at[0, :, 0],
                              ssems[h]).wait()

    fire_g(0, 0)
    drain_g(0)
    fire_g(1, 1)
    transpose(0, 0)
    fire_s(0, 0)
    drain_g(1)
    fire_g(2, 0)
    transpose(1, 1)
    fire_s(1, 1)

    @pl.loop(2, _PER_W - 2, step=2)
    def _(t0):
        for p in range(2):
            t = t0 + p
            h = p
            drain_g(h)
            fire_g(t + 1, 1 - h)
            drain_s(h)
            transpose(t, h)
            fire_s(t, h)

    drain_g(0)
    fire_g(_PER_W - 1, 1)
    drain_s(0)
    transpose(_PER_W - 2, 0)
    fire_s(_PER_W - 2, 0)
    drain_g(1)
    drain_s(1)
    transpose(_PER_W - 1, 1)
    fire_s(_PER_W - 1, 1)
    drain_s(0)
    drain_s(1)


def kernel(word_indices, table):
    idx = word_indices.T.reshape(_NW, _PER_W, 128).astype(jnp.int32)
    tab2 = table.reshape(500000, 128)
    out = _emb(idx, tab2)
    return out.transpose(2, 4, 0, 1, 3).reshape(_B, _L, _D)


# speedup vs baseline: 1.1163x; 1.1163x over previous
import functools

import jax
import jax.numpy as jnp
from jax import lax
from jax.experimental import pallas as pl
from jax.experimental.pallas import tpu as pltpu
from jax.experimental.pallas import tpu_sc as plsc

_B, _L, _D = 4096, 200, 64
_NC, _NS = 2, 16
_NW = _NC * _NS
_NBLK = _L * (_B // 128)
_PER_W = _NBLK // _NW
_BH = _B // 128

_mesh = plsc.VectorSubcoreMesh(core_axis_name="c", subcore_axis_name="s")


@functools.partial(
    pl.kernel,
    mesh=_mesh,
    out_type=jax.ShapeDtypeStruct((_L, 8, _BH, 8, 128), jnp.float32),
    scratch_types=[
        pltpu.VMEM((_PER_W, 128), jnp.int32),
        pltpu.VMEM((_PER_W, 128), jnp.int32),
        pltpu.VMEM((2, 128, 128), jnp.float32),
        pltpu.VMEM((2, 8, 8, 128), jnp.float32),
        pltpu.SemaphoreType.DMA,
        pltpu.SemaphoreType.DMA,
        pltpu.SemaphoreType.DMA,
    ],
    compiler_params=pltpu.CompilerParams(use_tc_tiling_on_sc=True,
                                         needs_layout_passes=False),
)
def _emb(idx_hbm, tab_hbm, out_hbm, pidx_v, pov_v, rows_v, trows_v, gsem,
         ssem0, ssem1):
    wid = lax.axis_index("s") * _NC + lax.axis_index("c")
    base = wid * _PER_W
    pltpu.sync_copy(idx_hbm.at[wid], pidx_v)

    @pl.loop(0, _PER_W)
    def _(t):
        for j in range(8):
            v = pidx_v[t, pl.ds(j * 16, 16)]
            pov_v[t, pl.ds(j * 16, 16)] = (v & 1) * 64
            pidx_v[t, pl.ds(j * 16, 16)] = v >> 1

    ssems = (ssem0, ssem1)
    lanes = lax.iota(jnp.int32, 16)
    ris = [b0 * 16 + lanes for b0 in range(8)]
    rots = [(lanes + k) % 16 for k in range(16)]

    def fire_g(t, h):
        pltpu.async_copy(tab_hbm.at[pidx_v.at[t]], rows_v.at[h], gsem)

    def drain_g(h):
        pltpu.make_async_copy(tab_hbm.at[pidx_v.at[0]], rows_v.at[h],
                              gsem).wait()

    def transpose(t, h):
        rv = rows_v.at[h]
        tv = trows_v.at[h]
        povs = [pov_v[t, pl.ds(b0 * 16, 16)] for b0 in range(8)]

        @pl.loop(0, 4)
        def _(q):
            d0 = q * 16
            for k in range(16):
                cik = rots[k] + d0
                ih = cik >> 3
                il = cik & 7
                for b0 in range(8):
                    v = plsc.load_gather(rv, [ris[b0], cik + povs[b0]])
                    plsc.store_scatter(tv, [ih, il, ris[b0]], v)

    def fire_s(t, h):
        blk = base + t
        l = blk // _BH
        bh = blk % _BH
        pltpu.async_copy(trows_v.at[h], out_hbm.at[l, :, bh], ssems[h])

    def drain_s(h):
        pltpu.make_async_copy(trows_v.at[h], out_hbm.at[0, :, 0],
                              ssems[h]).wait()

    fire_g(0, 0)
    drain_g(0)
    fire_g(1, 1)
    transpose(0, 0)
    fire_s(0, 0)
    drain_g(1)
    fire_g(2, 0)
    transpose(1, 1)
    fire_s(1, 1)

    @pl.loop(2, _PER_W - 2, step=2)
    def _(t0):
        for p in range(2):
            t = t0 + p
            h = p
            drain_g(h)
            fire_g(t + 1, 1 - h)
            drain_s(h)
            transpose(t, h)
            fire_s(t, h)

    drain_g(0)
    fire_g(_PER_W - 1, 1)
    drain_s(0)
    transpose(_PER_W - 2, 0)
    fire_s(_PER_W - 2, 0)
    drain_g(1)
    drain_s(1)
    transpose(_PER_W - 1, 1)
    fire_s(_PER_W - 1, 1)
    drain_s(0)
    drain_s(1)


def kernel(word_indices, table):
    idx = word_indices.T.reshape(_NW, _PER_W, 128).astype(jnp.int32)
    tab2 = table.reshape(500000, 128)
    out = _emb(idx, tab2)
    return out.transpose(2, 4, 0, 1, 3).reshape(_B, _L, _D)


# speedup vs baseline: 1.2057x; 1.0801x over previous
import functools

import jax
import jax.numpy as jnp
from jax import lax
from jax.experimental import pallas as pl
from jax.experimental.pallas import tpu as pltpu
from jax.experimental.pallas import tpu_sc as plsc

_B, _L, _D = 4096, 200, 64
_NC, _NS = 2, 16
_NW = _NC * _NS
_NBLK = _L * (_B // 128)
_PER_W = _NBLK // _NW
_BH = _B // 128

_mesh = plsc.VectorSubcoreMesh(core_axis_name="c", subcore_axis_name="s")


@functools.partial(
    pl.kernel,
    mesh=_mesh,
    out_type=jax.ShapeDtypeStruct((_L, 8, _BH, 8, 128), jnp.float32),
    scratch_types=[
        pltpu.VMEM((_PER_W, 128), jnp.int32),
        pltpu.VMEM((_PER_W, 128), jnp.int32),
        pltpu.VMEM((2, 128, 128), jnp.float32),
        pltpu.VMEM((2, 8, 8, 128), jnp.float32),
        pltpu.SemaphoreType.DMA,
        pltpu.SemaphoreType.DMA,
        pltpu.SemaphoreType.DMA,
    ],
    compiler_params=pltpu.CompilerParams(use_tc_tiling_on_sc=True,
                                         needs_layout_passes=False),
)
def _emb(idx_hbm, tab_hbm, out_hbm, pidx_v, pov_v, rows_v, trows_v, gsem,
         ssem0, ssem1):
    wid = lax.axis_index("s") * _NC + lax.axis_index("c")
    base = wid * _PER_W
    pltpu.sync_copy(idx_hbm.at[wid], pidx_v)

    @pl.loop(0, _PER_W)
    def _(t):
        for j in range(8):
            v = pidx_v[t, pl.ds(j * 16, 16)]
            pov_v[t, pl.ds(j * 16, 16)] = (v & 1) * 64
            pidx_v[t, pl.ds(j * 16, 16)] = v >> 1

    ssems = (ssem0, ssem1)
    lanes = lax.iota(jnp.int32, 16)
    ris = [b0 * 16 + lanes for b0 in range(8)]
    rots = [(lanes + k) % 16 for k in range(16)]

    def fire_g(t, h):
        pltpu.async_copy(tab_hbm.at[pidx_v.at[t]], rows_v.at[h], gsem)

    def drain_g(h):
        pltpu.make_async_copy(tab_hbm.at[pidx_v.at[0]], rows_v.at[h],
                              gsem).wait()

    def transpose(t, h):
        rv = rows_v.at[h]
        tv = trows_v.at[h]
        povs = [pov_v[t, pl.ds(b0 * 16, 16)] for b0 in range(8)]

        @pl.loop(0, 4)
        def _(q):
            d0 = q * 16

            def loads(k):
                cik = rots[k] + d0
                return [plsc.load_gather(rv, [ris[b0], cik + povs[b0]])
                        for b0 in range(8)]

            def stores(k, vs):
                cik = rots[k] + d0
                ih = cik >> 3
                il = cik & 7
                for b0 in range(8):
                    plsc.store_scatter(tv, [ih, il, ris[b0]], vs[b0])

            prev = loads(0)
            for k in range(1, 16):
                cur = loads(k)
                stores(k - 1, prev)
                prev = cur
            stores(15, prev)

    def fire_s(t, h):
        blk = base + t
        l = blk // _BH
        bh = blk % _BH
        pltpu.async_copy(trows_v.at[h], out_hbm.at[l, :, bh], ssems[h])

    def drain_s(h):
        pltpu.make_async_copy(trows_v.at[h], out_hbm.at[0, :, 0],
                              ssems[h]).wait()

    fire_g(0, 0)
    drain_g(0)
    fire_g(1, 1)
    transpose(0, 0)
    fire_s(0, 0)
    drain_g(1)
    fire_g(2, 0)
    transpose(1, 1)
    fire_s(1, 1)

    @pl.loop(2, _PER_W - 2, step=2)
    def _(t0):
        for p in range(2):
            t = t0 + p
            h = p
            drain_g(h)
            fire_g(t + 1, 1 - h)
            drain_s(h)
            transpose(t, h)
            fire_s(t, h)

    drain_g(0)
    fire_g(_PER_W - 1, 1)
    drain_s(0)
    transpose(_PER_W - 2, 0)
    fire_s(_PER_W - 2, 0)
    drain_g(1)
    drain_s(1)
    transpose(_PER_W - 1, 1)
    fire_s(_PER_W - 1, 1)
    drain_s(0)
    drain_s(1)


def kernel(word_indices, table):
    idx = word_indices.T.reshape(_NW, _PER_W, 128).astype(jnp.int32)
    tab2 = table.reshape(500000, 128)
    out = _emb(idx, tab2)
    return out.transpose(2, 4, 0, 1, 3).reshape(_B, _L, _D)


# speedup vs baseline: 1.2059x; 1.0001x over previous
import functools

import jax
import jax.numpy as jnp
from jax import lax
from jax.experimental import pallas as pl
from jax.experimental.pallas import tpu as pltpu
from jax.experimental.pallas import tpu_sc as plsc

_B, _L, _D = 4096, 200, 64
_NC, _NS = 2, 16
_NW = _NC * _NS
_NBLK = _L * (_B // 128)
_PER_W = _NBLK // _NW
_BH = _B // 128

_mesh = plsc.VectorSubcoreMesh(core_axis_name="c", subcore_axis_name="s")


@functools.partial(
    pl.kernel,
    mesh=_mesh,
    out_type=jax.ShapeDtypeStruct((_L, 8, _BH, 8, 128), jnp.float32),
    scratch_types=[
        pltpu.VMEM((_PER_W, 128), jnp.int32),
        pltpu.VMEM((_PER_W, 128), jnp.int32),
        pltpu.VMEM((2, 128, 128), jnp.float32),
        pltpu.VMEM((2, 8, 8, 128), jnp.float32),
        pltpu.SemaphoreType.DMA,
        pltpu.SemaphoreType.DMA,
        pltpu.SemaphoreType.DMA,
    ],
    compiler_params=pltpu.CompilerParams(use_tc_tiling_on_sc=True,
                                         needs_layout_passes=False),
)
def _emb(idx_hbm, tab_hbm, out_hbm, pidx_v, pov_v, rows_v, trows_v, gsem,
         ssem0, ssem1):
    wid = lax.axis_index("s") * _NC + lax.axis_index("c")
    base = wid * _PER_W
    pltpu.sync_copy(idx_hbm.at[wid], pidx_v)

    @pl.loop(0, _PER_W)
    def _(t):
        for j in range(8):
            v = pidx_v[t, pl.ds(j * 16, 16)]
            pov_v[t, pl.ds(j * 16, 16)] = (v & 1) * 64
            pidx_v[t, pl.ds(j * 16, 16)] = v >> 1

    ssems = (ssem0, ssem1)
    lanes = lax.iota(jnp.int32, 16)
    ris = [b0 * 16 + lanes for b0 in range(8)]
    rots = [(lanes + k) % 16 for k in range(16)]

    def fire_g(t, h):
        pltpu.async_copy(tab_hbm.at[pidx_v.at[t]], rows_v.at[h], gsem)

    def drain_g(h):
        pltpu.make_async_copy(tab_hbm.at[pidx_v.at[0]], rows_v.at[h],
                              gsem).wait()

    def transpose(t, h):
        rv = rows_v.at[h]
        tv = trows_v.at[h]
        povs = [pov_v[t, pl.ds(b0 * 16, 16)] for b0 in range(8)]

        @pl.loop(0, 4)
        def _(q):
            d0 = q * 16

            def loads(k):
                cik = rots[k] + d0
                return [plsc.load_gather(rv, [ris[b0], cik + povs[b0]])
                        for b0 in range(8)]

            def stores(k, vs):
                cik = rots[k] + d0
                ih = cik >> 3
                il = cik & 7
                for b0 in range(8):
                    plsc.store_scatter(tv, [ih, il, ris[b0]], vs[b0])

            p0 = loads(0)
            p1 = loads(1)
            for k in range(2, 16):
                cur = loads(k)
                stores(k - 2, p0)
                p0, p1 = p1, cur
            stores(14, p0)
            stores(15, p1)

    def fire_s(t, h):
        blk = base + t
        l = blk // _BH
        bh = blk % _BH
        pltpu.async_copy(trows_v.at[h], out_hbm.at[l, :, bh], ssems[h])

    def drain_s(h):
        pltpu.make_async_copy(trows_v.at[h], out_hbm.at[0, :, 0],
                              ssems[h]).wait()

    fire_g(0, 0)
    drain_g(0)
    fire_g(1, 1)
    transpose(0, 0)
    fire_s(0, 0)
    drain_g(1)
    fire_g(2, 0)
    transpose(1, 1)
    fire_s(1, 1)

    @pl.loop(2, _PER_W - 2, step=2)
    def _(t0):
        for p in range(2):
            t = t0 + p
            h = p
            drain_g(h)
            fire_g(t + 1, 1 - h)
            drain_s(h)
            transpose(t, h)
            fire_s(t, h)

    drain_g(0)
    fire_g(_PER_W - 1, 1)
    drain_s(0)
    transpose(_PER_W - 2, 0)
    fire_s(_PER_W - 2, 0)
    drain_g(1)
    drain_s(1)
    transpose(_PER_W - 1, 1)
    fire_s(_PER_W - 1, 1)
    drain_s(0)
    drain_s(1)


def kernel(word_indices, table):
    idx = word_indices.T.reshape(_NW, _PER_W, 128).astype(jnp.int32)
    tab2 = table.reshape(500000, 128)
    out = _emb(idx, tab2)
    return out.transpose(2, 4, 0, 1, 3).reshape(_B, _L, _D)
